# baseline (speedup 1.0000x reference)
import functools

import jax
import jax.numpy as jnp
from jax import lax
from jax.experimental import pallas as pl
from jax.experimental.pallas import tpu as pltpu
from jax.experimental.pallas import tpu_sc as plsc

B = 16
MAX_SEQLEN = 4096
TOTAL = 32768
CLS_ID = 101
SEP_ID = 102
OUT_LEN = MAX_SEQLEN + 2
HALF = 2048
NPIECE = HALF // 128
CHUNKS_LOOP = HALF // 16
WIN = HALF + 16
STAGE = 24
BASE_MAX = TOTAL - WIN
SHIFT_MAX = TOTAL + HALF - BASE_MAX
IN_V = STAGE + SHIFT_MAX + HALF + 16


def _row_body(flat_hbm, cu_hbm, out_hbm, cu_v, in_v, out_v, sem):
    c = lax.axis_index("c")
    s = lax.axis_index("s")

    pltpu.sync_copy(cu_hbm, cu_v.at[pl.ds(0, 17)])
    iota = lax.broadcasted_iota(jnp.int32, (16,), 0)
    start = cu_v[pl.ds(s, 16)][0]
    end = cu_v[pl.ds(s + 1, 16)][0]
    ln = end - start

    col0 = c * HALF
    a = start + col0
    base = jnp.clip(jnp.bitwise_and(a - 8, -8), 0, BASE_MAX)
    base = pl.multiple_of(base, 8)
    shift = a - base
    pltpu.sync_copy(flat_hbm.at[pl.ds(base, WIN)], in_v.at[pl.ds(STAGE, WIN)])

    @pl.when(c == 0)
    def _():
        in_v[pl.ds(STAGE - 16 + shift, 16)] = jnp.where(iota == 15, CLS_ID, 0)
    sep_at = STAGE - 1 + shift + (ln + 1 - col0)
    in_sep = (ln + 1 >= col0) & (ln + 1 < col0 + HALF)
    sep_off = jnp.where(in_sep, sep_at, 0)

    @pl.when(in_sep)
    def _():
        old = in_v[pl.ds(sep_off, 16)]
        in_v[pl.ds(sep_off, 16)] = jnp.where(iota == 0, SEP_ID, old)

    lim = ln + 1 - col0

    @plsc.parallel_loop(0, HALF, step=16, unroll=4)
    def _(i):
        vals = in_v[pl.ds(STAGE - 1 + shift + i, 16)]
        keep = iota + i <= lim
        out_v[pl.ds(i, 16)] = jnp.where(keep, vals, 0)

    copies = [
        pltpu.async_copy(
            out_v.at[pl.ds(t * 128, 128)],
            out_hbm.at[s, pl.ds(col0 + t * 128, 128)],
            sem,
        )
        for t in range(NPIECE)
    ]
    for cp in copies:
        cp.wait()


@functools.partial(
    pl.kernel,
    out_type=jax.ShapeDtypeStruct((B, OUT_LEN), jnp.int32),
    mesh=plsc.VectorSubcoreMesh(core_axis_name="c", subcore_axis_name="s"),
    compiler_params=pltpu.CompilerParams(
        use_tc_tiling_on_sc=True, skip_device_barrier=True
    ),
    scratch_types=[
        pltpu.VMEM((32,), jnp.int32),
        pltpu.VMEM((IN_V,), jnp.int32),
        pltpu.VMEM((HALF,), jnp.int32),
        pltpu.SemaphoreType.DMA,
    ],
)
def _sc_merge(flat_hbm, cu_hbm, out_hbm, cu_v, in_v, out_v, sem):
    _row_body(flat_hbm, cu_hbm, out_hbm, cu_v, in_v, out_v, sem)


def kernel(flat_ids, cu_seqlens):
    cu = cu_seqlens.astype(jnp.int32)
    out = _sc_merge(flat_ids, cu)
    ln = cu[1:] - cu[:-1]
    last_tok = flat_ids[jnp.clip(cu[1:] - 1, 0, TOTAL - 1)]
    c0 = jnp.where(ln == MAX_SEQLEN, last_tok,
                   jnp.where(ln == MAX_SEQLEN - 1, SEP_ID, 0))
    c1 = jnp.where(ln == MAX_SEQLEN, SEP_ID, 0)
    tail = jnp.stack([c0, c1], axis=1).astype(out.dtype)
    return lax.dynamic_update_slice(out, tail, (0, MAX_SEQLEN))


# speedup vs baseline: 1.0497x; 1.0497x over previous
import functools

import jax
import jax.numpy as jnp
from jax import lax
from jax.experimental import pallas as pl
from jax.experimental.pallas import tpu as pltpu
from jax.experimental.pallas import tpu_sc as plsc

B = 16
MAX_SEQLEN = 4096
TOTAL = 32768
CLS_ID = 101
SEP_ID = 102
OUT_LEN = MAX_SEQLEN + 2
FULLW = 4096
NPIECE = FULLW // 128
WIN = FULLW + 16
STAGE = 24
BASE_MAX = TOTAL - WIN
SHIFT_MAX = TOTAL - BASE_MAX
IN_V = STAGE + SHIFT_MAX + FULLW + 16


def _row_body(flat_hbm, cu_hbm, out_hbm, cu_v, in_v, out_v, sem):
    s = lax.axis_index("s")

    pltpu.sync_copy(cu_hbm, cu_v.at[pl.ds(0, 17)])
    iota = lax.broadcasted_iota(jnp.int32, (16,), 0)
    start = cu_v[pl.ds(s, 16)][0]
    end = cu_v[pl.ds(s + 1, 16)][0]
    ln = end - start

    base = jnp.clip(jnp.bitwise_and(start - 8, -8), 0, BASE_MAX)
    base = pl.multiple_of(base, 8)
    shift = start - base
    pltpu.sync_copy(flat_hbm.at[pl.ds(base, WIN)], in_v.at[pl.ds(STAGE, WIN)])

    in_v[pl.ds(STAGE - 16 + shift, 16)] = jnp.where(iota == 15, CLS_ID, 0)
    sep_at = STAGE - 1 + shift + ln + 1
    in_sep = ln + 1 < FULLW
    sep_off = jnp.where(in_sep, sep_at, 0)

    @pl.when(in_sep)
    def _():
        old = in_v[pl.ds(sep_off, 16)]
        in_v[pl.ds(sep_off, 16)] = jnp.where(iota == 0, SEP_ID, old)

    lim = ln + 1

    @plsc.parallel_loop(0, FULLW, step=16, unroll=4)
    def _(i):
        vals = in_v[pl.ds(STAGE - 1 + shift + i, 16)]
        keep = iota + i <= lim
        out_v[pl.ds(i, 16)] = jnp.where(keep, vals, 0)

    copies = [
        pltpu.async_copy(
            out_v.at[pl.ds(t * 128, 128)],
            out_hbm.at[s, pl.ds(t * 128, 128)],
            sem,
        )
        for t in range(NPIECE)
    ]
    for cp in copies:
        cp.wait()


@functools.partial(
    pl.kernel,
    out_type=jax.ShapeDtypeStruct((B, OUT_LEN), jnp.int32),
    mesh=plsc.VectorSubcoreMesh(
        core_axis_name="c", subcore_axis_name="s", num_cores=1
    ),
    compiler_params=pltpu.CompilerParams(use_tc_tiling_on_sc=True),
    scratch_types=[
        pltpu.VMEM((32,), jnp.int32),
        pltpu.VMEM((IN_V,), jnp.int32),
        pltpu.VMEM((FULLW,), jnp.int32),
        pltpu.SemaphoreType.DMA,
    ],
)
def _sc_merge(flat_hbm, cu_hbm, out_hbm, cu_v, in_v, out_v, sem):
    _row_body(flat_hbm, cu_hbm, out_hbm, cu_v, in_v, out_v, sem)


def kernel(flat_ids, cu_seqlens):
    cu = cu_seqlens.astype(jnp.int32)
    out = _sc_merge(flat_ids, cu)
    ln = cu[1:] - cu[:-1]
    last_tok = flat_ids[jnp.clip(cu[1:] - 1, 0, TOTAL - 1)]
    c0 = jnp.where(ln == MAX_SEQLEN, last_tok,
                   jnp.where(ln == MAX_SEQLEN - 1, SEP_ID, 0))
    c1 = jnp.where(ln == MAX_SEQLEN, SEP_ID, 0)
    tail = jnp.stack([c0, c1], axis=1).astype(out.dtype)
    return lax.dynamic_update_slice(out, tail, (0, MAX_SEQLEN))


# speedup vs baseline: 1.0526x; 1.0027x over previous
import functools

import jax
import jax.numpy as jnp
from jax import lax
from jax.experimental import pallas as pl
from jax.experimental.pallas import tpu as pltpu
from jax.experimental.pallas import tpu_sc as plsc

B = 16
MAX_SEQLEN = 4096
TOTAL = 32768
CLS_ID = 101
SEP_ID = 102
OUT_LEN = MAX_SEQLEN + 2
FULLW = 4096
HALFW = FULLW // 2
NPIECE_H = HALFW // 128
WIN_A = HALFW + 16
WIN_B = HALFW
STAGE = 24
IN_V = STAGE + 16 + FULLW + 16
PAD_TOTAL = TOTAL + MAX_SEQLEN


def _row_body(flat_hbm, cu_hbm, out_hbm, cu_v, in_v, out_v, sem_a, sem_b, sem_o):
    s = lax.axis_index("s")

    pltpu.sync_copy(cu_hbm, cu_v.at[pl.ds(0, 17)])
    iota = lax.broadcasted_iota(jnp.int32, (16,), 0)
    start = cu_v[pl.ds(s, 16)][0]
    end = cu_v[pl.ds(s + 1, 16)][0]
    ln = end - start
    lim = ln + 1

    base = jnp.maximum(jnp.bitwise_and(start - 8, -8), 0)
    base = pl.multiple_of(base, 8)
    shift = start - base
    cp_a = pltpu.async_copy(
        flat_hbm.at[pl.ds(base, WIN_A)], in_v.at[pl.ds(STAGE, WIN_A)], sem_a
    )
    cp_b = pltpu.async_copy(
        flat_hbm.at[pl.ds(base + WIN_A, WIN_B)],
        in_v.at[pl.ds(STAGE + WIN_A, WIN_B)],
        sem_b,
    )
    cp_a.wait()

    in_v[pl.ds(STAGE - 16 + shift, 16)] = jnp.where(iota == 15, CLS_ID, 0)
    sep_at = STAGE - 1 + shift + ln + 1

    def plant_sep(cond):
        sep_off = jnp.where(cond, sep_at, 0)

        @pl.when(cond)
        def _():
            old = in_v[pl.ds(sep_off, 16)]
            in_v[pl.ds(sep_off, 16)] = jnp.where(iota == 0, SEP_ID, old)

    plant_sep(ln + 1 <= HALFW - 1)

    @plsc.parallel_loop(0, HALFW, step=16, unroll=4)
    def _(i):
        vals = in_v[pl.ds(STAGE - 1 + shift + i, 16)]
        keep = iota + i <= lim
        out_v[pl.ds(i, 16)] = jnp.where(keep, vals, 0)

    copies = [
        pltpu.async_copy(
            out_v.at[pl.ds(t * 128, 128)],
            out_hbm.at[s, pl.ds(t * 128, 128)],
            sem_o,
        )
        for t in range(NPIECE_H)
    ]

    cp_b.wait()
    plant_sep((ln + 1 >= HALFW) & (ln + 1 <= FULLW - 1))

    @plsc.parallel_loop(HALFW, FULLW, step=16, unroll=4)
    def _(i):
        vals = in_v[pl.ds(STAGE - 1 + shift + i, 16)]
        keep = iota + i <= lim
        out_v[pl.ds(i, 16)] = jnp.where(keep, vals, 0)

    copies += [
        pltpu.async_copy(
            out_v.at[pl.ds(HALFW + t * 128, 128)],
            out_hbm.at[s, pl.ds(HALFW + t * 128, 128)],
            sem_o,
        )
        for t in range(NPIECE_H)
    ]
    for cp in copies:
        cp.wait()


@functools.partial(
    pl.kernel,
    out_type=jax.ShapeDtypeStruct((B, OUT_LEN), jnp.int32),
    mesh=plsc.VectorSubcoreMesh(
        core_axis_name="c", subcore_axis_name="s", num_cores=1
    ),
    compiler_params=pltpu.CompilerParams(use_tc_tiling_on_sc=True),
    scratch_types=[
        pltpu.VMEM((32,), jnp.int32),
        pltpu.VMEM((IN_V,), jnp.int32),
        pltpu.VMEM((FULLW,), jnp.int32),
        pltpu.SemaphoreType.DMA,
        pltpu.SemaphoreType.DMA,
        pltpu.SemaphoreType.DMA,
    ],
)
def _sc_merge(flat_hbm, cu_hbm, out_hbm, cu_v, in_v, out_v, sem_a, sem_b, sem_o):
    _row_body(flat_hbm, cu_hbm, out_hbm, cu_v, in_v, out_v, sem_a, sem_b, sem_o)


def kernel(flat_ids, cu_seqlens):
    cu = cu_seqlens.astype(jnp.int32)
    flat_pad = jnp.pad(flat_ids, (0, PAD_TOTAL - TOTAL))
    out = _sc_merge(flat_pad, cu)
    ln = cu[1:] - cu[:-1]
    last_tok = flat_ids[jnp.clip(cu[1:] - 1, 0, TOTAL - 1)]
    c0 = jnp.where(ln == MAX_SEQLEN, last_tok,
                   jnp.where(ln == MAX_SEQLEN - 1, SEP_ID, 0))
    c1 = jnp.where(ln == MAX_SEQLEN, SEP_ID, 0)
    tail = jnp.stack([c0, c1], axis=1).astype(out.dtype)
    return lax.dynamic_update_slice(out, tail, (0, MAX_SEQLEN))
